# baseline (speedup 1.0000x reference)
import functools

import jax
import jax.numpy as jnp
from jax.experimental import pallas as pl
from jax.experimental.pallas import tpu as pltpu


def _round_up(x, m):
    return (x + m - 1) // m * m


def _ffn_block_kernel(h_ref, fnw_ref, w13_ref, w2_ref, anw_ref,
                      o_ref, x_ref, g_ref, *, eps, inv_dim):
    k = pl.program_id(1)
    n = pl.num_programs(1)
    th = w2_ref.shape[0]

    @pl.when(k == 0)
    def _init():
        h = h_ref[...]
        ms = jnp.sum(h * h, axis=-1, keepdims=True) * inv_dim
        x_ref[...] = (h * jax.lax.rsqrt(ms + eps) * fnw_ref[...]).astype(x_ref.dtype)
        o_ref[...] = h

    @pl.when(k > 0)
    def _down_proj():
        o_ref[...] += jnp.dot(g_ref[...], w2_ref[...],
                              preferred_element_type=jnp.float32)

    @pl.when(k < n - 1)
    def _gate_up():
        hh = jnp.dot(x_ref[...], w13_ref[...], preferred_element_type=jnp.float32)
        g_ref[...] = (jax.nn.silu(hh[:, :th]) * hh[:, th:]).astype(g_ref.dtype)

    @pl.when(k == n - 1)
    def _finalize():
        y = o_ref[...]
        ms2 = jnp.sum(y * y, axis=-1, keepdims=True) * inv_dim
        o_ref[...] = y * jax.lax.rsqrt(ms2 + eps) * anw_ref[...]


def kernel(h, ffn_nw, w13, w2, attn_nw, *, eps=1e-6):
    B, S, dim = h.shape
    dim_p = ffn_nw.shape[1]
    th = 256
    nk = w13.shape[1] // (2 * th)
    tokens = B * S

    tm = 512
    while tokens % tm and tm > 8:
        tm //= 2
    tokens_p = _round_up(tokens, tm)
    n_tiles = tokens_p // tm

    h2d = h.reshape(tokens, dim)
    if tokens_p != tokens or dim_p != dim:
        h2d = jnp.pad(h2d, ((0, tokens_p - tokens), (0, dim_p - dim)))

    w_bytes = (w13.size + w2.size) * w13.dtype.itemsize
    cost = pl.CostEstimate(
        flops=int(6 * tokens_p * dim_p * nk * th),
        transcendentals=int(tokens_p * nk * th + 2 * tokens_p),
        bytes_accessed=int(w_bytes * n_tiles + 2 * tokens_p * dim_p * 4),
    )

    body = functools.partial(_ffn_block_kernel, eps=eps, inv_dim=1.0 / dim)

    out = pl.pallas_call(
        body,
        out_shape=jax.ShapeDtypeStruct((tokens_p, dim_p), h.dtype),
        grid=(n_tiles, nk + 1),
        in_specs=[
            pl.BlockSpec((tm, dim_p), lambda i, k: (i, 0)),
            pl.BlockSpec((1, dim_p), lambda i, k: (0, 0)),
            pl.BlockSpec((dim_p, 2 * th),
                         lambda i, k: (0, jnp.minimum(k, pl.num_programs(1) - 2))),
            pl.BlockSpec((th, dim_p),
                         lambda i, k: (jnp.maximum(k - 1, 0), 0)),
            pl.BlockSpec((1, dim_p), lambda i, k: (0, 0)),
        ],
        out_specs=pl.BlockSpec((tm, dim_p), lambda i, k: (i, 0)),
        scratch_shapes=[
            pltpu.VMEM((tm, dim_p), w13.dtype),
            pltpu.VMEM((tm, th), w13.dtype),
        ],
        compiler_params=pltpu.CompilerParams(
            dimension_semantics=("parallel", "arbitrary"),
            vmem_limit_bytes=60 * 1024 * 1024,
        ),
        cost_estimate=cost,
    )(h2d, ffn_nw, w13, w2, attn_nw)

    if tokens_p != tokens or dim_p != dim:
        out = out[:tokens, :dim]
    return out.reshape(B, S, dim)


# speedup vs baseline: 1.1098x; 1.1098x over previous
import functools

import jax
import jax.numpy as jnp
from jax.experimental import pallas as pl
from jax.experimental.pallas import tpu as pltpu


def _round_up(x, m):
    return (x + m - 1) // m * m


def _ffn_block_kernel(h_ref, fnw_ref, w13_ref, w2_ref, anw_ref,
                      o_ref, x_ref, *, eps, inv_dim, row_chunk):
    k = pl.program_id(1)
    th = w2_ref.shape[0]
    tm = o_ref.shape[0]
    n_chunks = tm // row_chunk

    @pl.when(k == 0)
    def _init():
        fnw = fnw_ref[...]

        def body(c, _):
            rows = pl.ds(c * row_chunk, row_chunk)
            hc = h_ref[rows, :]
            ms = jnp.sum(hc * hc, axis=-1, keepdims=True) * inv_dim
            x_ref[rows, :] = (hc * jax.lax.rsqrt(ms + eps) * fnw).astype(x_ref.dtype)
            o_ref[rows, :] = hc
            return 0

        jax.lax.fori_loop(0, n_chunks, body, 0)

    hh = jnp.dot(x_ref[...], w13_ref[...], preferred_element_type=jnp.float32)
    gated = jax.nn.silu(hh[:, :th]) * hh[:, th:]
    o_ref[...] += jnp.dot(gated.astype(w2_ref.dtype), w2_ref[...],
                          preferred_element_type=jnp.float32)

    @pl.when(k == pl.num_programs(1) - 1)
    def _finalize():
        anw = anw_ref[...]

        def body(c, _):
            rows = pl.ds(c * row_chunk, row_chunk)
            y = o_ref[rows, :]
            ms2 = jnp.sum(y * y, axis=-1, keepdims=True) * inv_dim
            o_ref[rows, :] = y * jax.lax.rsqrt(ms2 + eps) * anw
            return 0

        jax.lax.fori_loop(0, n_chunks, body, 0)


def kernel(h, ffn_nw, w13, w2, attn_nw, *, eps=1e-6):
    B, S, dim = h.shape
    dim_p = ffn_nw.shape[1]
    th = 256
    nk = w13.shape[1] // (2 * th)
    tokens = B * S

    tm = 1024
    while tokens % tm and tm > 8:
        tm //= 2
    tokens_p = _round_up(tokens, tm)
    n_tiles = tokens_p // tm
    row_chunk = min(256, tm)

    h2d = h.reshape(tokens, dim)
    if tokens_p != tokens or dim_p != dim:
        h2d = jnp.pad(h2d, ((0, tokens_p - tokens), (0, dim_p - dim)))

    w_bytes = (w13.size + w2.size) * w13.dtype.itemsize
    cost = pl.CostEstimate(
        flops=int(6 * tokens_p * dim_p * nk * th),
        transcendentals=int(tokens_p * nk * th + 2 * tokens_p),
        bytes_accessed=int(w_bytes * n_tiles + 2 * tokens_p * dim_p * 4),
    )

    body = functools.partial(_ffn_block_kernel, eps=eps, inv_dim=1.0 / dim,
                             row_chunk=row_chunk)

    out = pl.pallas_call(
        body,
        out_shape=jax.ShapeDtypeStruct((tokens_p, dim_p), h.dtype),
        grid=(n_tiles, nk),
        in_specs=[
            pl.BlockSpec((tm, dim_p), lambda i, k: (i, 0),
                         pipeline_mode=pl.Buffered(buffer_count=1)),
            pl.BlockSpec((1, dim_p), lambda i, k: (0, 0)),
            pl.BlockSpec((dim_p, 2 * th), lambda i, k: (0, k)),
            pl.BlockSpec((th, dim_p), lambda i, k: (k, 0)),
            pl.BlockSpec((1, dim_p), lambda i, k: (0, 0)),
        ],
        out_specs=pl.BlockSpec((tm, dim_p), lambda i, k: (i, 0),
                               pipeline_mode=pl.Buffered(buffer_count=1)),
        scratch_shapes=[pltpu.VMEM((tm, dim_p), w13.dtype)],
        compiler_params=pltpu.CompilerParams(
            dimension_semantics=("parallel", "arbitrary"),
            vmem_limit_bytes=62 * 1024 * 1024,
        ),
        cost_estimate=cost,
    )(h2d, ffn_nw, w13, w2, attn_nw)

    if tokens_p != tokens or dim_p != dim:
        out = out[:tokens, :dim]
    return out.reshape(B, S, dim)


# speedup vs baseline: 1.1132x; 1.0031x over previous
import functools

import jax
import jax.numpy as jnp
from jax.experimental import pallas as pl
from jax.experimental.pallas import tpu as pltpu


def _round_up(x, m):
    return (x + m - 1) // m * m


def _ffn_block_kernel(h_ref, fnw_ref, w13_ref, w2_ref, anw_ref,
                      o_ref, x_ref, *, eps, inv_dim, row_chunk):
    k = pl.program_id(1)
    th = w2_ref.shape[0]
    tm = o_ref.shape[0]
    n_chunks = tm // row_chunk

    @pl.when(k == 0)
    def _init():
        fnw = fnw_ref[...]

        def body(c, _):
            rows = pl.ds(c * row_chunk, row_chunk)
            hc = h_ref[rows, :]
            ms = jnp.sum(hc * hc, axis=-1, keepdims=True) * inv_dim
            x_ref[rows, :] = (hc * jax.lax.rsqrt(ms + eps) * fnw).astype(x_ref.dtype)
            o_ref[rows, :] = hc
            return 0

        jax.lax.fori_loop(0, n_chunks, body, 0)

    half = tm // 2
    for rows in (pl.ds(0, half), pl.ds(half, half)):
        hh = jnp.dot(x_ref[rows, :], w13_ref[...],
                     preferred_element_type=jnp.float32)
        gated = jax.nn.silu(hh[:, :th]) * hh[:, th:]
        o_ref[rows, :] += jnp.dot(gated.astype(w2_ref.dtype), w2_ref[...],
                                  preferred_element_type=jnp.float32)

    @pl.when(k == pl.num_programs(1) - 1)
    def _finalize():
        anw = anw_ref[...]

        def body(c, _):
            rows = pl.ds(c * row_chunk, row_chunk)
            y = o_ref[rows, :]
            ms2 = jnp.sum(y * y, axis=-1, keepdims=True) * inv_dim
            o_ref[rows, :] = y * jax.lax.rsqrt(ms2 + eps) * anw
            return 0

        jax.lax.fori_loop(0, n_chunks, body, 0)


def kernel(h, ffn_nw, w13, w2, attn_nw, *, eps=1e-6):
    B, S, dim = h.shape
    dim_p = ffn_nw.shape[1]
    th = 256
    nk = w13.shape[1] // (2 * th)
    tokens = B * S

    tm = 1024
    while tokens % tm and tm > 8:
        tm //= 2
    tokens_p = _round_up(tokens, tm)
    n_tiles = tokens_p // tm
    row_chunk = min(256, tm)

    h2d = h.reshape(tokens, dim)
    if tokens_p != tokens or dim_p != dim:
        h2d = jnp.pad(h2d, ((0, tokens_p - tokens), (0, dim_p - dim)))

    w_bytes = (w13.size + w2.size) * w13.dtype.itemsize
    cost = pl.CostEstimate(
        flops=int(6 * tokens_p * dim_p * nk * th),
        transcendentals=int(tokens_p * nk * th + 2 * tokens_p),
        bytes_accessed=int(w_bytes * n_tiles + 2 * tokens_p * dim_p * 4),
    )

    body = functools.partial(_ffn_block_kernel, eps=eps, inv_dim=1.0 / dim,
                             row_chunk=row_chunk)

    out = pl.pallas_call(
        body,
        out_shape=jax.ShapeDtypeStruct((tokens_p, dim_p), h.dtype),
        grid=(n_tiles, nk),
        in_specs=[
            pl.BlockSpec((tm, dim_p), lambda i, k: (i, 0),
                         pipeline_mode=pl.Buffered(buffer_count=1)),
            pl.BlockSpec((1, dim_p), lambda i, k: (0, 0)),
            pl.BlockSpec((dim_p, 2 * th), lambda i, k: (0, k)),
            pl.BlockSpec((th, dim_p), lambda i, k: (k, 0)),
            pl.BlockSpec((1, dim_p), lambda i, k: (0, 0)),
        ],
        out_specs=pl.BlockSpec((tm, dim_p), lambda i, k: (i, 0),
                               pipeline_mode=pl.Buffered(buffer_count=1)),
        scratch_shapes=[pltpu.VMEM((tm, dim_p), w13.dtype)],
        compiler_params=pltpu.CompilerParams(
            dimension_semantics=("parallel", "arbitrary"),
            vmem_limit_bytes=62 * 1024 * 1024,
        ),
        cost_estimate=cost,
    )(h2d, ffn_nw, w13, w2, attn_nw)

    if tokens_p != tokens or dim_p != dim:
        out = out[:tokens, :dim]
    return out.reshape(B, S, dim)
